# baseline (speedup 1.0000x reference)
import jax
import jax.numpy as jnp
from jax import lax
from jax.experimental import pallas as pl
from jax.experimental.pallas import tpu as pltpu
from jax.experimental.pallas import tpu_sc as plsc

N = 8192
D = 2048
E = 16
BN = 1024

NC = 2
NS = 16
NW = NC * NS
RW = N // NW
G = RW // 16


def _dense_block(h_ref, ww_ref, wn_ref, bw_ref, bn_ref, eps_ref,
                 noisyt_ref, full_ref):
    h = h_ref[...]
    dims = (((1,), (1,)), ((), ()))
    logits = lax.dot_general(h, ww_ref[...], dims,
                             preferred_element_type=jnp.float32) + bw_ref[...]
    zn = lax.dot_general(h, wn_ref[...], dims,
                         preferred_element_type=jnp.float32) + bn_ref[...]
    noisy = logits + eps_ref[...] * jax.nn.softplus(zn)
    noisyt_ref[...] = noisy.T
    m = jnp.max(noisy, axis=-1, keepdims=True)
    e = jnp.exp(noisy - m)
    full_ref[...] = e / jnp.sum(e, axis=-1, keepdims=True)


def _route_sc(noisyt_hbm, probst_hbm, ixt_hbm, noisy_v, probs_v, ix_v):
    wid = lax.axis_index("s") * NC + lax.axis_index("c")
    base = wid * RW
    pltpu.sync_copy(noisyt_hbm.at[:, pl.ds(base, RW)], noisy_v)

    jvecs = [jnp.full((16,), j, jnp.int32) for j in range(E)]
    neg_inf = jnp.full((16,), -jnp.inf, jnp.float32)
    zeros_f = jnp.zeros((16,), jnp.float32)

    def group(g, _):
        c = g * 16
        v = [noisy_v[j, pl.ds(c, 16)] for j in range(E)]

        m1 = v[0]
        for j in range(1, E):
            m1 = jnp.maximum(m1, v[j])
        i1 = jnp.full((16,), E, jnp.int32)
        for j in range(E):
            i1 = jnp.minimum(i1, jnp.where(v[j] == m1, jvecs[j], E))

        m2 = neg_inf
        for j in range(E):
            m2 = jnp.maximum(m2, jnp.where(i1 == jvecs[j], neg_inf, v[j]))
        i2 = jnp.full((16,), E, jnp.int32)
        for j in range(E):
            hit = (v[j] == m2) & (i1 != jvecs[j])
            i2 = jnp.minimum(i2, jnp.where(hit, jvecs[j], E))

        t = jnp.exp(m2 - m1)
        r = 1.0 / (1.0 + t)
        p1 = r
        p2 = t * r
        for j in range(E):
            probs_v[j, pl.ds(c, 16)] = jnp.where(
                i1 == jvecs[j], p1, jnp.where(i2 == jvecs[j], p2, zeros_f))
        ix_v[0, pl.ds(c, 16)] = i1
        ix_v[1, pl.ds(c, 16)] = i2
        return ()

    lax.fori_loop(0, G, group, ())

    pltpu.sync_copy(probs_v, probst_hbm.at[:, pl.ds(base, RW)])
    pltpu.sync_copy(ix_v, ixt_hbm.at[:, pl.ds(base, RW)])


@jax.jit
def kernel(h, Ww, bw, Wn, bn, eps):
    bw2 = bw.reshape(1, E)
    bn2 = bn.reshape(1, E)
    noisyt, full = pl.pallas_call(
        _dense_block,
        grid=(N // BN,),
        in_specs=[
            pl.BlockSpec((BN, D), lambda i: (i, 0)),
            pl.BlockSpec((E, D), lambda i: (0, 0)),
            pl.BlockSpec((E, D), lambda i: (0, 0)),
            pl.BlockSpec((1, E), lambda i: (0, 0)),
            pl.BlockSpec((1, E), lambda i: (0, 0)),
            pl.BlockSpec((BN, E), lambda i: (i, 0)),
        ],
        out_specs=[
            pl.BlockSpec((E, BN), lambda i: (0, i)),
            pl.BlockSpec((BN, E), lambda i: (i, 0)),
        ],
        out_shape=[
            jax.ShapeDtypeStruct((E, N), jnp.float32),
            jax.ShapeDtypeStruct((N, E), jnp.float32),
        ],
    )(h, Ww, Wn, bw2, bn2, eps)

    route = pl.kernel(
        _route_sc,
        mesh=plsc.VectorSubcoreMesh(core_axis_name="c", subcore_axis_name="s"),
        compiler_params=pltpu.CompilerParams(needs_layout_passes=False),
        out_type=[
            jax.ShapeDtypeStruct((E, N), jnp.float32),
            jax.ShapeDtypeStruct((2, N), jnp.int32),
        ],
        scratch_types=[
            pltpu.VMEM((E, RW), jnp.float32),
            pltpu.VMEM((E, RW), jnp.float32),
            pltpu.VMEM((2, RW), jnp.int32),
        ],
    )
    probst, ixt = route(noisyt)
    return probst.T, ixt.T, full


# speedup vs baseline: 1.0148x; 1.0148x over previous
import jax
import jax.numpy as jnp
from jax import lax
from jax.experimental import pallas as pl
from jax.experimental.pallas import tpu as pltpu
from jax.experimental.pallas import tpu_sc as plsc

N = 8192
D = 2048
E = 16
BN = 1024

NC = 2
NS = 16
NW = NC * NS
RW = N // NW
G = RW // 16


def _dense_block(h_ref, ww_ref, wn_ref, bw_ref, bn_ref, eps_ref,
                 noisyt_ref, fullt_ref):
    h = h_ref[...]
    dims = (((1,), (1,)), ((), ()))
    logits = lax.dot_general(h, ww_ref[...], dims,
                             preferred_element_type=jnp.float32) + bw_ref[...]
    zn = lax.dot_general(h, wn_ref[...], dims,
                         preferred_element_type=jnp.float32) + bn_ref[...]
    noisy = logits + eps_ref[...] * jax.nn.softplus(zn)
    noisyt_ref[...] = noisy.T
    m = jnp.max(noisy, axis=-1, keepdims=True)
    e = jnp.exp(noisy - m)
    fullt_ref[...] = (e / jnp.sum(e, axis=-1, keepdims=True)).T


def _route_sc(noisyt_hbm, probst_hbm, ixt_hbm, noisy_v, probs_v, ix_v):
    wid = lax.axis_index("s") * NC + lax.axis_index("c")
    base = wid * RW
    pltpu.sync_copy(noisyt_hbm.at[:, pl.ds(base, RW)], noisy_v)

    jvecs = [jnp.full((16,), j, jnp.int32) for j in range(E)]
    neg_inf = jnp.full((16,), -jnp.inf, jnp.float32)
    zeros_f = jnp.zeros((16,), jnp.float32)

    def group(g, _):
        c = g * 16
        v = [noisy_v[j, pl.ds(c, 16)] for j in range(E)]

        m1 = v[0]
        for j in range(1, E):
            m1 = jnp.maximum(m1, v[j])
        i1 = jnp.full((16,), E, jnp.int32)
        for j in range(E):
            i1 = jnp.minimum(i1, jnp.where(v[j] == m1, jvecs[j], E))

        m2 = neg_inf
        for j in range(E):
            m2 = jnp.maximum(m2, jnp.where(i1 == jvecs[j], neg_inf, v[j]))
        i2 = jnp.full((16,), E, jnp.int32)
        for j in range(E):
            hit = (v[j] == m2) & (i1 != jvecs[j])
            i2 = jnp.minimum(i2, jnp.where(hit, jvecs[j], E))

        t = jnp.exp(m2 - m1)
        r = 1.0 / (1.0 + t)
        p1 = r
        p2 = t * r
        for j in range(E):
            probs_v[j, pl.ds(c, 16)] = jnp.where(
                i1 == jvecs[j], p1, jnp.where(i2 == jvecs[j], p2, zeros_f))
        ix_v[0, pl.ds(c, 16)] = i1
        ix_v[1, pl.ds(c, 16)] = i2
        return ()

    lax.fori_loop(0, G, group, ())

    pltpu.sync_copy(probs_v, probst_hbm.at[:, pl.ds(base, RW)])
    pltpu.sync_copy(ix_v, ixt_hbm.at[:, pl.ds(base, RW)])


@jax.jit
def kernel(h, Ww, bw, Wn, bn, eps):
    bw2 = bw.reshape(1, E)
    bn2 = bn.reshape(1, E)

    noisyt, fullt = pl.pallas_call(
        _dense_block,
        grid=(N // BN,),
        in_specs=[
            pl.BlockSpec((BN, D), lambda i: (i, 0)),
            pl.BlockSpec((E, D), lambda i: (0, 0)),
            pl.BlockSpec((E, D), lambda i: (0, 0)),
            pl.BlockSpec((1, E), lambda i: (0, 0)),
            pl.BlockSpec((1, E), lambda i: (0, 0)),
            pl.BlockSpec((BN, E), lambda i: (i, 0)),
        ],
        out_specs=[
            pl.BlockSpec((E, BN), lambda i: (0, i)),
            pl.BlockSpec((E, BN), lambda i: (0, i)),
        ],
        out_shape=[
            jax.ShapeDtypeStruct((E, N), jnp.float32),
            jax.ShapeDtypeStruct((E, N), jnp.float32),
        ],
        compiler_params=pltpu.CompilerParams(
            dimension_semantics=("arbitrary",)),
    )(h, Ww, Wn, bw2, bn2, eps)

    route = pl.kernel(
        _route_sc,
        mesh=plsc.VectorSubcoreMesh(core_axis_name="c", subcore_axis_name="s"),
        compiler_params=pltpu.CompilerParams(needs_layout_passes=False),
        out_type=[
            jax.ShapeDtypeStruct((E, N), jnp.float32),
            jax.ShapeDtypeStruct((2, N), jnp.int32),
        ],
        scratch_types=[
            pltpu.VMEM((E, RW), jnp.float32),
            pltpu.VMEM((E, RW), jnp.float32),
            pltpu.VMEM((2, RW), jnp.int32),
        ],
    )
    probst, ixt = route(noisyt)
    return probst.T, ixt.T, fullt.T
